# baseline (speedup 1.0000x reference)
import functools

import jax
import jax.numpy as jnp
from jax import lax
from jax.experimental import pallas as pl
from jax.experimental.pallas import tpu as pltpu
from jax.experimental.pallas import tpu_sc as plsc

B, T = 1024, 50
V, E, H = 100000, 128, 256
G = 3 * H


@functools.lru_cache(maxsize=1)
def _make_sc_gather():
    NC, NS = 2, 16
    NW = NC * NS
    TB = T * B
    per_w = TB // NW
    CH = 80
    n_ch = per_w // CH

    mesh = plsc.VectorSubcoreMesh(core_axis_name="c", subcore_axis_name="s")

    @functools.partial(
        pl.kernel,
        mesh=mesh,
        out_type=jax.ShapeDtypeStruct((TB, E), jnp.float32),
        scratch_types=[
            pltpu.VMEM((per_w,), jnp.int32),
            pltpu.VMEM((CH, E), jnp.float32),
            pltpu.VMEM((CH, E), jnp.float32),
            pltpu.SemaphoreType.DMA,
            pltpu.SemaphoreType.DMA,
        ],
    )
    def gather_k(table_hbm, idx_hbm, out_hbm, idx_v, rows0, rows1, s0, s1):
        wid = lax.axis_index("s") * NC + lax.axis_index("c")
        base = wid * per_w
        pltpu.sync_copy(idx_hbm.at[pl.ds(base, per_w)], idx_v)

        def fire(i, buf, sem):
            pltpu.async_copy(table_hbm.at[idx_v.at[pl.ds(i * CH, CH)]], buf, sem)

        def drain(buf, sem):
            pltpu.make_async_copy(table_hbm.at[idx_v.at[pl.ds(0, CH)]], buf, sem).wait()

        fire(0, rows0, s0)
        fire(1, rows1, s1)

        def body(j, carry):
            i0 = j * 2
            drain(rows0, s0)
            pltpu.sync_copy(rows0, out_hbm.at[pl.ds(base + i0 * CH, CH)])

            @pl.when(i0 + 2 < n_ch)
            def _():
                fire(i0 + 2, rows0, s0)

            drain(rows1, s1)
            pltpu.sync_copy(rows1, out_hbm.at[pl.ds(base + (i0 + 1) * CH, CH)])

            @pl.when(i0 + 3 < n_ch)
            def _():
                fire(i0 + 3, rows1, s1)

            return carry

        lax.fori_loop(0, n_ch // 2, body, 0)

    return gather_k


def _gru_body(e_ref, wih0, whh0, bih0, bhh0, wih1, whh1, bih1, bhh1,
              y_ref, hid_ref, h0_s, h1_s):
    t = pl.program_id(0)

    @pl.when(t == 0)
    def _():
        h0_s[...] = jnp.zeros_like(h0_s)
        h1_s[...] = jnp.zeros_like(h1_s)

    def cell(xt, h, wihT, whhT, bih, bhh):
        gi = lax.dot_general(xt, wihT, (((1,), (0,)), ((), ())),
                             precision=lax.Precision.DEFAULT,
                             preferred_element_type=jnp.float32) + bih
        gh = lax.dot_general(h, whhT, (((1,), (0,)), ((), ())),
                             precision=lax.Precision.DEFAULT,
                             preferred_element_type=jnp.float32) + bhh
        r = jax.nn.sigmoid(gi[:, :H] + gh[:, :H])
        z = jax.nn.sigmoid(gi[:, H:2 * H] + gh[:, H:2 * H])
        n = jnp.tanh(gi[:, 2 * H:] + r * gh[:, 2 * H:])
        return (1.0 - z) * n + z * h

    h0 = cell(e_ref[0], h0_s[...], wih0[...], whh0[...], bih0[...], bhh0[...])
    h0_s[...] = h0
    h1 = cell(h0, h1_s[...], wih1[...], whh1[...], bih1[...], bhh1[...])
    h1_s[...] = h1
    tm = t % 8
    for k in range(8):
        @pl.when(tm == k)
        def _(k=k):
            y_ref[:, k, :] = h1

    @pl.when(t == T - 1)
    def _():
        hid_ref[0] = h0
        hid_ref[1] = h1


def _gru2(e_tbE, wih0T, whh0T, bih0, bhh0, wih1T, whh1T, bih1, bhh1):
    full = lambda shape: pl.BlockSpec(shape, lambda t: tuple(0 for _ in shape))
    y, hid = pl.pallas_call(
        _gru_body,
        grid=(T,),
        in_specs=[
            pl.BlockSpec((1, B, E), lambda t: (t, 0, 0)),
            full((E, G)), full((H, G)), full((1, G)), full((1, G)),
            full((H, G)), full((H, G)), full((1, G)), full((1, G)),
        ],
        out_specs=[
            pl.BlockSpec((B, 8, H), lambda t: (0, t // 8, 0)),
            pl.BlockSpec((2, B, H), lambda t: (0, 0, 0)),
        ],
        out_shape=[
            jax.ShapeDtypeStruct((B, T, H), jnp.float32),
            jax.ShapeDtypeStruct((2, B, H), jnp.float32),
        ],
        scratch_shapes=[
            pltpu.VMEM((B, H), jnp.float32),
            pltpu.VMEM((B, H), jnp.float32),
        ],
    )(e_tbE, wih0T, whh0T, bih0, bhh0, wih1T, whh1T, bih1, bhh1)
    return y, hid


def kernel(x, emb, W_ih_l0, W_hh_l0, b_ih_l0, b_hh_l0,
           W_ih_l1, W_hh_l1, b_ih_l1, b_hh_l1):
    idx = x.T.reshape(-1).astype(jnp.int32)
    e = _make_sc_gather()(emb, idx)
    e = e.reshape(T, B, E)

    y, hid = _gru2(
        e,
        W_ih_l0.T, W_hh_l0.T, b_ih_l0.reshape(1, G), b_hh_l0.reshape(1, G),
        W_ih_l1.T, W_hh_l1.T, b_ih_l1.reshape(1, G), b_hh_l1.reshape(1, G),
    )
    return y, hid


# speedup vs baseline: 1.3879x; 1.3879x over previous
import functools

import jax
import jax.numpy as jnp
from jax import lax
from jax.experimental import pallas as pl
from jax.experimental.pallas import tpu as pltpu
from jax.experimental.pallas import tpu_sc as plsc

B, T = 1024, 50
V, E, H = 100000, 128, 256
G = 3 * H


@functools.lru_cache(maxsize=1)
def _make_sc_gather():
    NC, NS = 2, 16
    NW = NC * NS
    TB = T * B
    per_w = TB // NW
    CH = 80
    n_ch = per_w // CH

    mesh = plsc.VectorSubcoreMesh(core_axis_name="c", subcore_axis_name="s")

    @functools.partial(
        pl.kernel,
        mesh=mesh,
        out_type=jax.ShapeDtypeStruct((TB, E), jnp.float32),
        scratch_types=[
            pltpu.VMEM((per_w,), jnp.int32),
            pltpu.VMEM((CH, E), jnp.float32),
            pltpu.VMEM((CH, E), jnp.float32),
            pltpu.SemaphoreType.DMA,
            pltpu.SemaphoreType.DMA,
        ],
    )
    def gather_k(table_hbm, idx_hbm, out_hbm, idx_v, rows0, rows1, s0, s1):
        wid = lax.axis_index("s") * NC + lax.axis_index("c")
        base = wid * per_w
        pltpu.sync_copy(idx_hbm.at[pl.ds(base, per_w)], idx_v)

        def fire(i, buf, sem):
            pltpu.async_copy(table_hbm.at[idx_v.at[pl.ds(i * CH, CH)]], buf, sem)

        def drain(buf, sem):
            pltpu.make_async_copy(table_hbm.at[idx_v.at[pl.ds(0, CH)]], buf, sem).wait()

        fire(0, rows0, s0)
        fire(1, rows1, s1)

        def body(j, carry):
            i0 = j * 2
            drain(rows0, s0)
            pltpu.sync_copy(rows0, out_hbm.at[pl.ds(base + i0 * CH, CH)])

            @pl.when(i0 + 2 < n_ch)
            def _():
                fire(i0 + 2, rows0, s0)

            drain(rows1, s1)
            pltpu.sync_copy(rows1, out_hbm.at[pl.ds(base + (i0 + 1) * CH, CH)])

            @pl.when(i0 + 3 < n_ch)
            def _():
                fire(i0 + 3, rows1, s1)

            return carry

        lax.fori_loop(0, n_ch // 2, body, 0)

    return gather_k


def _gru_body(e_ref, wih0, whh0, bih0, bhh0, wih1, whh1, bih1, bhh1,
              y_ref, hid_ref, h0_s, h1_s):
    t = pl.program_id(0)

    @pl.when(t == 0)
    def _():
        h0_s[...] = jnp.zeros_like(h0_s)
        h1_s[...] = jnp.zeros_like(h1_s)

    def cell(xt, h, wihT, whhT, bih, bhh):
        gi = lax.dot_general(xt, wihT, (((1,), (0,)), ((), ())),
                             precision=lax.Precision.DEFAULT,
                             preferred_element_type=jnp.float32) + bih
        gh = lax.dot_general(h, whhT, (((1,), (0,)), ((), ())),
                             precision=lax.Precision.DEFAULT,
                             preferred_element_type=jnp.float32) + bhh
        r = jax.nn.sigmoid(gi[:, :H] + gh[:, :H])
        z = jax.nn.sigmoid(gi[:, H:2 * H] + gh[:, H:2 * H])
        n = jnp.tanh(gi[:, 2 * H:] + r * gh[:, 2 * H:])
        return (1.0 - z) * n + z * h

    h0 = cell(e_ref[0], h0_s[...], wih0[...], whh0[...], bih0[...], bhh0[...])
    h0_s[...] = h0
    h1 = cell(h0, h1_s[...], wih1[...], whh1[...], bih1[...], bhh1[...])
    h1_s[...] = h1
    y_ref[0] = h1

    @pl.when(t == T - 1)
    def _():
        hid_ref[0] = h0
        hid_ref[1] = h1


def _gru2(e_tbE, wih0T, whh0T, bih0, bhh0, wih1T, whh1T, bih1, bhh1):
    full = lambda shape: pl.BlockSpec(shape, lambda t: tuple(0 for _ in shape))
    y, hid = pl.pallas_call(
        _gru_body,
        grid=(T,),
        in_specs=[
            pl.BlockSpec((1, B, E), lambda t: (t, 0, 0)),
            full((E, G)), full((H, G)), full((1, G)), full((1, G)),
            full((H, G)), full((H, G)), full((1, G)), full((1, G)),
        ],
        out_specs=[
            pl.BlockSpec((1, B, H), lambda t: (t, 0, 0)),
            pl.BlockSpec((2, B, H), lambda t: (0, 0, 0)),
        ],
        out_shape=[
            jax.ShapeDtypeStruct((T, B, H), jnp.float32),
            jax.ShapeDtypeStruct((2, B, H), jnp.float32),
        ],
        scratch_shapes=[
            pltpu.VMEM((B, H), jnp.float32),
            pltpu.VMEM((B, H), jnp.float32),
        ],
    )(e_tbE, wih0T, whh0T, bih0, bhh0, wih1T, whh1T, bih1, bhh1)
    return y, hid


def kernel(x, emb, W_ih_l0, W_hh_l0, b_ih_l0, b_hh_l0,
           W_ih_l1, W_hh_l1, b_ih_l1, b_hh_l1):
    idx = x.T.reshape(-1).astype(jnp.int32)
    e = _make_sc_gather()(emb, idx)
    e = e.reshape(T, B, E)

    y, hid = _gru2(
        e,
        W_ih_l0.T, W_hh_l0.T, b_ih_l0.reshape(1, G), b_hh_l0.reshape(1, G),
        W_ih_l1.T, W_hh_l1.T, b_ih_l1.reshape(1, G), b_hh_l1.reshape(1, G),
    )
    return jnp.swapaxes(y, 0, 1), hid
